# baseline (speedup 1.0000x reference)
import jax
import jax.numpy as jnp
from jax import lax
from jax.experimental import pallas as pl
from jax.experimental.pallas import tpu as pltpu
from jax.experimental.pallas import tpu_sc as plsc

NUM_ITEMS = 1000000
BATCH = 16384
HIST = 200
N = BATCH * HIST

NC = 2
NS = 16
NW = NC * NS
L = 16

ROWS_PER_W = BATCH // NW
CROWS = 64
CHUNK = CROWS * HIST
N_CHUNKS = ROWS_PER_W // CROWS
N_PIECES_ROW = (HIST + L - 1) // L
STAGE_PIECE = 10000
N_PIECES = NUM_ITEMS // STAGE_PIECE


def _gather_kernel(table_hbm, idx_hbm, out_hbm, tab_s,
                   slab_v, flat0, flat1, rows0, rows1, gsem, ssem):
    cid = lax.axis_index("c")
    sid = lax.axis_index("s")
    wid = sid * NC + cid
    base_row = wid * ROWS_PER_W
    base = base_row * HIST

    flat = (flat0, flat1)
    rows = (rows0, rows1)
    lanes = lax.iota(jnp.int32, L)

    def load_bridge(b, c):
        row0 = base_row + c * CROWS
        pltpu.sync_copy(idx_hbm.at[pl.ds(row0, CROWS)], slab_v)

        def r_body(r, carry):
            dst0 = r * HIST
            for k in range(N_PIECES_ROW):
                off = min(k * L, HIST - L)
                v = slab_v[r, pl.ds(off, L)]
                flat[b][pl.ds(dst0 + off, L)] = v
            return carry

        lax.fori_loop(0, CROWS, r_body, 0)

    def sbody(p, carry):
        @pl.when(lax.rem(p, NS) == sid)
        def _():
            off = pl.multiple_of(p * STAGE_PIECE, 8)
            pltpu.sync_copy(table_hbm.at[pl.ds(off, STAGE_PIECE)],
                            rows0.at[pl.ds(0, STAGE_PIECE)])
            pltpu.sync_copy(rows0.at[pl.ds(0, STAGE_PIECE)],
                            tab_s.at[pl.ds(off, STAGE_PIECE)])
        return carry

    lax.fori_loop(0, N_PIECES, sbody, 0)

    load_bridge(0, 0)
    load_bridge(1, 1)
    plsc.subcore_barrier()

    def pair_body(i, carry):
        c0 = 2 * i
        ga = pltpu.async_copy(tab_s.at[flat[0]], rows[0], gsem)
        gb = pltpu.async_copy(tab_s.at[flat[1]], rows[1], gsem)
        ga.wait()
        s = pltpu.async_copy(
            rows[0], out_hbm.at[pl.ds(pl.multiple_of(base + c0 * CHUNK, 8),
                                      CHUNK)], ssem)
        load_bridge(0, lax.rem(c0 + 2, N_CHUNKS))
        s.wait()
        gb.wait()
        s = pltpu.async_copy(
            rows[1], out_hbm.at[pl.ds(pl.multiple_of(base + (c0 + 1) * CHUNK,
                                                     8), CHUNK)], ssem)
        load_bridge(1, lax.rem(c0 + 3, N_CHUNKS))
        s.wait()
        return carry

    lax.fori_loop(0, N_CHUNKS // 2, pair_body, 0)


@jax.jit
def kernel(item_ids, score_embedding):
    mesh = plsc.VectorSubcoreMesh(core_axis_name="c", subcore_axis_name="s")
    out = pl.kernel(
        _gather_kernel,
        mesh=mesh,
        out_type=jax.ShapeDtypeStruct((N,), jnp.float32),
        scratch_types=[
            pltpu.VMEM_SHARED((NUM_ITEMS,), jnp.float32),
            pltpu.VMEM((CROWS, HIST), jnp.int32),
            pltpu.VMEM((CHUNK,), jnp.int32),
            pltpu.VMEM((CHUNK,), jnp.int32),
            pltpu.VMEM((CHUNK,), jnp.float32),
            pltpu.VMEM((CHUNK,), jnp.float32),
            pltpu.SemaphoreType.DMA,
            pltpu.SemaphoreType.DMA,
        ],
    )(score_embedding.reshape(NUM_ITEMS), item_ids)
    return out.reshape(BATCH, HIST, 1)


# speedup vs baseline: 1.1149x; 1.1149x over previous
import jax
import jax.numpy as jnp
from jax import lax
from jax.experimental import pallas as pl
from jax.experimental.pallas import tpu as pltpu
from jax.experimental.pallas import tpu_sc as plsc

NUM_ITEMS = 1000000
BATCH = 16384
HIST = 200
N = BATCH * HIST

NC = 2
NS = 16
NW = NC * NS

N_PER_W = N // NW
CHUNK = 12800
N_CHUNKS = N_PER_W // CHUNK
STAGE_PIECE = 10000
N_PIECES = NUM_ITEMS // STAGE_PIECE


def _gather_kernel(table_hbm, idx_hbm, out_hbm, tab_s,
                   idx0, idx1, rows0, rows1, gsem, ssem):
    cid = lax.axis_index("c")
    sid = lax.axis_index("s")
    wid = sid * NC + cid
    base = wid * N_PER_W

    def coff(c):
        return pl.multiple_of(base + c * CHUNK, CHUNK)

    def sbody(p, carry):
        @pl.when(lax.rem(p, NS) == sid)
        def _():
            off = pl.multiple_of(p * STAGE_PIECE, 8)
            pltpu.sync_copy(table_hbm.at[pl.ds(off, STAGE_PIECE)],
                            rows0.at[pl.ds(0, STAGE_PIECE)])
            pltpu.sync_copy(rows0.at[pl.ds(0, STAGE_PIECE)],
                            tab_s.at[pl.ds(off, STAGE_PIECE)])
        return carry

    lax.fori_loop(0, N_PIECES, sbody, 0)

    idx = (idx0, idx1)
    rows = (rows0, rows1)
    pltpu.sync_copy(idx_hbm.at[pl.ds(coff(0), CHUNK)], idx[0])
    pltpu.sync_copy(idx_hbm.at[pl.ds(coff(1), CHUNK)], idx[1])
    plsc.subcore_barrier()

    gs = [pltpu.async_copy(tab_s.at[idx[0]], rows[0], gsem),
          pltpu.async_copy(tab_s.at[idx[1]], rows[1], gsem)]
    for c in range(N_CHUNKS):
        b = c & 1
        gs[b].wait()
        s = pltpu.async_copy(rows[b], out_hbm.at[pl.ds(coff(c), CHUNK)], ssem)
        if c + 2 < N_CHUNKS:
            pltpu.sync_copy(idx_hbm.at[pl.ds(coff(c + 2), CHUNK)], idx[b])
            s.wait()
            gs[b] = pltpu.async_copy(tab_s.at[idx[b]], rows[b], gsem)
        else:
            s.wait()


@jax.jit
def kernel(item_ids, score_embedding):
    idx = item_ids.reshape(N)
    table = score_embedding.reshape(NUM_ITEMS)
    mesh = plsc.VectorSubcoreMesh(core_axis_name="c", subcore_axis_name="s")
    out = pl.kernel(
        _gather_kernel,
        mesh=mesh,
        out_type=jax.ShapeDtypeStruct((N,), jnp.float32),
        scratch_types=[
            pltpu.VMEM_SHARED((NUM_ITEMS,), jnp.float32),
            pltpu.VMEM((CHUNK,), jnp.int32),
            pltpu.VMEM((CHUNK,), jnp.int32),
            pltpu.VMEM((CHUNK,), jnp.float32),
            pltpu.VMEM((CHUNK,), jnp.float32),
            pltpu.SemaphoreType.DMA,
            pltpu.SemaphoreType.DMA,
        ],
    )(table, idx)
    return out.reshape(BATCH, HIST, 1)


# speedup vs baseline: 1.1154x; 1.0004x over previous
import jax
import jax.numpy as jnp
from jax import lax
from jax.experimental import pallas as pl
from jax.experimental.pallas import tpu as pltpu
from jax.experimental.pallas import tpu_sc as plsc

NUM_ITEMS = 1000000
BATCH = 16384
HIST = 200
N = BATCH * HIST

NC = 2
NS = 16
NW = NC * NS

N_PER_W = N // NW
CHUNK = 12800
N_CHUNKS = N_PER_W // CHUNK
STAGE_PIECE = 10000
N_PIECES = NUM_ITEMS // STAGE_PIECE


def _gather_kernel(table_hbm, idx_hbm, out_hbm, tab_s,
                   idx0, idx1, rows0, rows1, gsem, ssem):
    cid = lax.axis_index("c")
    sid = lax.axis_index("s")
    wid = sid * NC + cid
    base = wid * N_PER_W

    def coff(c):
        return pl.multiple_of(base + c * CHUNK, CHUNK)

    def sbody(p, carry):
        @pl.when(lax.rem(p, NS) == sid)
        def _():
            off = pl.multiple_of(p * STAGE_PIECE, 8)
            pltpu.sync_copy(table_hbm.at[pl.ds(off, STAGE_PIECE)],
                            rows0.at[pl.ds(0, STAGE_PIECE)])
            pltpu.sync_copy(rows0.at[pl.ds(0, STAGE_PIECE)],
                            tab_s.at[pl.ds(off, STAGE_PIECE)])
        return carry

    lax.fori_loop(0, N_PIECES, sbody, 0)

    idx = (idx0, idx1)
    rows = (rows0, rows1)
    pltpu.sync_copy(idx_hbm.at[pl.ds(coff(0), CHUNK)], idx[0])
    pltpu.sync_copy(idx_hbm.at[pl.ds(coff(1), CHUNK)], idx[1])
    plsc.subcore_barrier()

    gs = [pltpu.async_copy(tab_s.at[idx[0]], rows[0], gsem),
          pltpu.async_copy(tab_s.at[idx[1]], rows[1], gsem)]
    for c in range(N_CHUNKS):
        b = c & 1
        gs[b].wait()
        s = pltpu.async_copy(rows[b], out_hbm.at[pl.ds(coff(c), CHUNK)], ssem)
        if c + 2 < N_CHUNKS:
            pltpu.sync_copy(idx_hbm.at[pl.ds(coff(c + 2), CHUNK)], idx[b])
            s.wait()
            gs[b] = pltpu.async_copy(tab_s.at[idx[b]], rows[b], gsem)
        else:
            s.wait()


@jax.jit
def kernel(item_ids, score_embedding):
    idx = item_ids.reshape(N)
    table = score_embedding.reshape(NUM_ITEMS)
    mesh = plsc.VectorSubcoreMesh(core_axis_name="c", subcore_axis_name="s")
    out = pl.kernel(
        _gather_kernel,
        mesh=mesh,
        compiler_params=pltpu.CompilerParams(use_tc_tiling_on_sc=True),
        out_type=jax.ShapeDtypeStruct((N,), jnp.float32),
        scratch_types=[
            pltpu.VMEM_SHARED((NUM_ITEMS,), jnp.float32),
            pltpu.VMEM((CHUNK,), jnp.int32),
            pltpu.VMEM((CHUNK,), jnp.int32),
            pltpu.VMEM((CHUNK,), jnp.float32),
            pltpu.VMEM((CHUNK,), jnp.float32),
            pltpu.SemaphoreType.DMA,
            pltpu.SemaphoreType.DMA,
        ],
    )(table, idx)
    return out.reshape(BATCH, HIST, 1)


# speedup vs baseline: 1.1154x; 1.0001x over previous
import jax
import jax.numpy as jnp
from jax import lax
from jax.experimental import pallas as pl
from jax.experimental.pallas import tpu as pltpu
from jax.experimental.pallas import tpu_sc as plsc

NUM_ITEMS = 1000000
BATCH = 16384
HIST = 200
N = BATCH * HIST

NC = 2
NS = 16
NW = NC * NS

N_PER_W = N // NW
CHUNK = 12800
N_CHUNKS = N_PER_W // CHUNK
STAGE_PIECE = 10000
N_PIECES = NUM_ITEMS // STAGE_PIECE


def _gather_kernel(table_hbm, idx_hbm, out_hbm, tab_s,
                   idx0, idx1, rows0, rows1, gsem, ssem):
    cid = lax.axis_index("c")
    sid = lax.axis_index("s")
    wid = sid * NC + cid
    base = wid * N_PER_W

    def coff(c):
        return pl.multiple_of(base + c * CHUNK, CHUNK)

    def sbody(p, carry):
        @pl.when(lax.rem(p, NS) == sid)
        def _():
            off = pl.multiple_of(p * STAGE_PIECE, 8)
            pltpu.sync_copy(table_hbm.at[pl.ds(off, STAGE_PIECE)],
                            rows0.at[pl.ds(0, STAGE_PIECE)])
            pltpu.sync_copy(rows0.at[pl.ds(0, STAGE_PIECE)],
                            tab_s.at[pl.ds(off, STAGE_PIECE)])
        return carry

    lax.fori_loop(0, N_PIECES, sbody, 0)

    idx = (idx0, idx1)
    rows = (rows0, rows1)
    pltpu.sync_copy(idx_hbm.at[pl.ds(coff(0), CHUNK)], idx[0])
    pltpu.sync_copy(idx_hbm.at[pl.ds(coff(1), CHUNK)], idx[1])
    plsc.subcore_barrier()

    gs = [pltpu.async_copy(tab_s.at[idx[0]], rows[0], gsem),
          pltpu.async_copy(tab_s.at[idx[1]], rows[1], gsem)]
    for c in range(N_CHUNKS):
        b = c & 1
        gs[b].wait()
        s = pltpu.async_copy(rows[b], out_hbm.at[pl.ds(coff(c), CHUNK)], ssem)
        if c + 2 < N_CHUNKS:
            pltpu.sync_copy(idx_hbm.at[pl.ds(coff(c + 2), CHUNK)], idx[b])
            s.wait()
            gs[b] = pltpu.async_copy(tab_s.at[idx[b]], rows[b], gsem)
        else:
            s.wait()


@jax.jit
def kernel(item_ids, score_embedding):
    idx = item_ids.reshape(N)
    table = score_embedding.T.reshape(NUM_ITEMS)
    mesh = plsc.VectorSubcoreMesh(core_axis_name="c", subcore_axis_name="s")
    out = pl.kernel(
        _gather_kernel,
        mesh=mesh,
        out_type=jax.ShapeDtypeStruct((N,), jnp.float32),
        scratch_types=[
            pltpu.VMEM_SHARED((NUM_ITEMS,), jnp.float32),
            pltpu.VMEM((CHUNK,), jnp.int32),
            pltpu.VMEM((CHUNK,), jnp.int32),
            pltpu.VMEM((CHUNK,), jnp.float32),
            pltpu.VMEM((CHUNK,), jnp.float32),
            pltpu.SemaphoreType.DMA,
            pltpu.SemaphoreType.DMA,
        ],
    )(table, idx)
    return out.reshape(BATCH, HIST, 1)
